# Initial kernel scaffold; baseline (speedup 1.0000x reference)
import jax
import jax.numpy as jnp
from jax.experimental import pallas as pl


def kernel(x, edge_index, W1, b1, W2, b2):
    raise NotImplementedError("write your pallas kernel here")


# speedup vs baseline: 15.5795x; 15.5795x over previous
import functools

import jax
import jax.numpy as jnp
from jax import lax
from jax.experimental import pallas as pl
from jax.experimental.pallas import tpu as pltpu
from jax.experimental.pallas import tpu_sc as plsc

N = 10000
NP = 10240
E = 320000
D_IN = 128
D_HID = 128
D_OUT = 64

NC = 2
NS = 16
LANES = 16
EPS = E // NS
K = 80
NCHUNK = EPS // K
HC = NCHUNK // NC
RPS = NP // NS
ZR = 128

BR = 2048


def _mesh():
    return plsc.VectorSubcoreMesh(
        core_axis_name="c", subcore_axis_name="s",
        num_cores=NC, num_subcores=NS)


_SC_PARAMS = pltpu.CompilerParams(use_tc_tiling_on_sc=False)


def _sc_degree(dst3):
    @functools.partial(
        pl.kernel,
        out_type=jax.ShapeDtypeStruct((NC, NP, LANES), jnp.float32),
        mesh=_mesh(),
        compiler_params=_SC_PARAMS,
        scratch_types=[
            pltpu.VMEM((NCHUNK, K), jnp.int32),
            pltpu.VMEM((K, LANES), jnp.float32),
            pltpu.VMEM((ZR, LANES), jnp.float32),
            pltpu.VMEM_SHARED((NP, LANES), jnp.float32),
        ],
    )
    def deg_kernel(dst_hbm, out_hbm, idx_v, ones_v, zero_v, acc_sh):
        cid = lax.axis_index("c")
        sid = lax.axis_index("s")

        one = jnp.full((LANES,), 1.0, jnp.float32)
        zero = jnp.zeros((LANES,), jnp.float32)

        @pl.loop(0, K)
        def _(i):
            ones_v[i, :] = one

        @pl.loop(0, ZR)
        def _(i):
            zero_v[i, :] = zero

        base = sid * RPS
        for t in range(RPS // ZR):
            pltpu.sync_copy(zero_v, acc_sh.at[pl.ds(base + t * ZR, ZR)])
        pltpu.sync_copy(dst_hbm.at[sid], idx_v)
        plsc.subcore_barrier()

        lo = cid * HC

        @pl.loop(0, HC)
        def _(j):
            pltpu.sync_copy(ones_v, acc_sh.at[idx_v.at[lo + j]], add=True)

        plsc.subcore_barrier()
        pltpu.sync_copy(acc_sh.at[pl.ds(base, RPS)],
                        out_hbm.at[cid, pl.ds(base, RPS)])

    return deg_kernel(dst3)


def _sc_aggregate(hs_split, src3, dst3, d2):
    @functools.partial(
        pl.kernel,
        out_type=jax.ShapeDtypeStruct((NC, NP, d2), jnp.float32),
        mesh=_mesh(),
        compiler_params=_SC_PARAMS,
        scratch_types=[
            pltpu.VMEM((NCHUNK, K), jnp.int32),
            pltpu.VMEM((NCHUNK, K), jnp.int32),
            pltpu.VMEM((K, d2), jnp.float32),
            pltpu.VMEM((ZR, d2), jnp.float32),
            pltpu.VMEM_SHARED((NP, d2), jnp.float32),
            pltpu.SemaphoreType.DMA,
        ],
    )
    def agg_kernel(hs_hbm, src_hbm, dst_hbm, out_hbm,
                   src_v, dst_v, rows_v, zero_v, acc_sh, sem):
        cid = lax.axis_index("c")
        sid = lax.axis_index("s")

        zero = jnp.zeros((LANES,), jnp.float32)

        @pl.loop(0, ZR)
        def _(i):
            @pl.loop(0, d2, step=LANES)
            def _(c0):
                zero_v[i, pl.ds(c0, LANES)] = zero

        base = sid * RPS
        for t in range(RPS // ZR):
            pltpu.sync_copy(zero_v, acc_sh.at[pl.ds(base + t * ZR, ZR)])
        pltpu.sync_copy(src_hbm.at[sid], src_v)
        pltpu.sync_copy(dst_hbm.at[sid], dst_v)
        plsc.subcore_barrier()

        @pl.loop(0, NCHUNK)
        def _(j):
            pltpu.async_copy(hs_hbm.at[cid].at[src_v.at[j]],
                             rows_v, sem).wait()
            pltpu.sync_copy(rows_v, acc_sh.at[dst_v.at[j]], add=True)

        plsc.subcore_barrier()
        pltpu.sync_copy(acc_sh.at[pl.ds(base, RPS)],
                        out_hbm.at[cid, pl.ds(base, RPS)])

    return agg_kernel(hs_split, src3, dst3)


def _dinv_of(degp_blk):
    deg = degp_blk[0] + degp_blk[1]
    return lax.rsqrt(deg[:, 0:1] + 1.0)


def _tc_matmul(x, w):
    m, k = x.shape
    n = w.shape[1]

    def body(x_ref, w_ref, o_ref):
        o_ref[...] = jnp.dot(x_ref[...], w_ref[...],
                             preferred_element_type=jnp.float32)

    return pl.pallas_call(
        body,
        grid=(m // BR,),
        in_specs=[pl.BlockSpec((BR, k), lambda i: (i, 0)),
                  pl.BlockSpec((k, n), lambda i: (0, 0))],
        out_specs=pl.BlockSpec((BR, n), lambda i: (i, 0)),
        out_shape=jax.ShapeDtypeStruct((m, n), jnp.float32),
    )(x, w)


def _tc_scale(h, degp):
    m, n = h.shape
    d2 = n // 2

    def body(h_ref, degp_ref, o_ref):
        hs = h_ref[...] * _dinv_of(degp_ref[...])
        o_ref[0] = hs[:, :d2]
        o_ref[1] = hs[:, d2:]

    return pl.pallas_call(
        body,
        grid=(m // BR,),
        in_specs=[pl.BlockSpec((BR, n), lambda i: (i, 0)),
                  pl.BlockSpec((NC, BR, LANES), lambda i: (0, i, 0))],
        out_specs=pl.BlockSpec((NC, BR, d2), lambda i: (0, i, 0)),
        out_shape=jax.ShapeDtypeStruct((NC, m, d2), jnp.float32),
    )(h, degp)


def _tc_mid(aggp, hs1s, degp, b1, w2):
    nc, m, d2 = hs1s.shape
    n = 2 * d2
    n2 = w2.shape[1]
    q2 = n2 // 2

    def body(a_ref, hs_ref, degp_ref, b_ref, w_ref, o_ref):
        dinv = _dinv_of(degp_ref[...])
        agg = jnp.concatenate([a_ref[0], a_ref[1]], axis=1)
        hs1 = jnp.concatenate([hs_ref[0], hs_ref[1]], axis=1)
        z = (agg + hs1) * dinv + b_ref[...]
        h2 = jnp.dot(jnp.maximum(z, 0.0), w_ref[...],
                     preferred_element_type=jnp.float32)
        hs2 = h2 * dinv
        o_ref[0] = hs2[:, :q2]
        o_ref[1] = hs2[:, q2:]

    return pl.pallas_call(
        body,
        grid=(m // BR,),
        in_specs=[pl.BlockSpec((NC, BR, d2), lambda i: (0, i, 0)),
                  pl.BlockSpec((NC, BR, d2), lambda i: (0, i, 0)),
                  pl.BlockSpec((NC, BR, LANES), lambda i: (0, i, 0)),
                  pl.BlockSpec((1, n), lambda i: (0, 0)),
                  pl.BlockSpec((n, n2), lambda i: (0, 0))],
        out_specs=pl.BlockSpec((NC, BR, q2), lambda i: (0, i, 0)),
        out_shape=jax.ShapeDtypeStruct((NC, m, q2), jnp.float32),
    )(aggp, hs1s, degp, b1, w2)


def _tc_fin(aggp, hs2s, degp, b2):
    nc, m, q2 = hs2s.shape
    n = 2 * q2

    def body(a_ref, hs_ref, degp_ref, b_ref, o_ref):
        dinv = _dinv_of(degp_ref[...])
        agg = jnp.concatenate([a_ref[0], a_ref[1]], axis=1)
        hs2 = jnp.concatenate([hs_ref[0], hs_ref[1]], axis=1)
        o_ref[...] = (agg + hs2) * dinv + b_ref[...]

    return pl.pallas_call(
        body,
        grid=(m // BR,),
        in_specs=[pl.BlockSpec((NC, BR, q2), lambda i: (0, i, 0)),
                  pl.BlockSpec((NC, BR, q2), lambda i: (0, i, 0)),
                  pl.BlockSpec((NC, BR, LANES), lambda i: (0, i, 0)),
                  pl.BlockSpec((1, n), lambda i: (0, 0))],
        out_specs=pl.BlockSpec((BR, n), lambda i: (i, 0)),
        out_shape=jax.ShapeDtypeStruct((m, n), jnp.float32),
    )(aggp, hs2s, degp, b2)


def kernel(x, edge_index, W1, b1, W2, b2):
    src = edge_index[0].astype(jnp.int32).reshape(NS, NCHUNK, K)
    dst = edge_index[1].astype(jnp.int32).reshape(NS, NCHUNK, K)
    xp = jnp.pad(x, ((0, NP - N), (0, 0)))

    degp = _sc_degree(dst)
    h1 = _tc_matmul(xp, W1)
    hs1s = _tc_scale(h1, degp)
    agg1 = _sc_aggregate(hs1s, src, dst, D_HID // 2)
    hs2s = _tc_mid(agg1, hs1s, degp, b1.reshape(1, D_HID), W2)
    agg2 = _sc_aggregate(hs2s, src, dst, D_OUT // 2)
    return _tc_fin(agg2, hs2s, degp, b2.reshape(1, D_OUT))[:N]


# speedup vs baseline: 23.0544x; 1.4798x over previous
import functools

import jax
import jax.numpy as jnp
from jax import lax
from jax.experimental import pallas as pl
from jax.experimental.pallas import tpu as pltpu
from jax.experimental.pallas import tpu_sc as plsc

N = 10000
NP = 10240
E = 320000
D_IN = 128
D_HID = 128
D_OUT = 64

NC = 2
NS = 16
LANES = 16
K = 128
NCHUNK = 158
EPS = NCHUNK * K
EP = NS * EPS
HC = NCHUNK // NC
RPS = NP // NS
ZR = 128

BR = 2048


def _mesh():
    return plsc.VectorSubcoreMesh(
        core_axis_name="c", subcore_axis_name="s",
        num_cores=NC, num_subcores=NS)


_SC_PARAMS = pltpu.CompilerParams(use_tc_tiling_on_sc=False)


def _sc_degree(dst3):
    @functools.partial(
        pl.kernel,
        out_type=jax.ShapeDtypeStruct((NC, NP, LANES), jnp.float32),
        mesh=_mesh(),
        compiler_params=_SC_PARAMS,
        scratch_types=[
            pltpu.VMEM((NCHUNK, K), jnp.int32),
            pltpu.VMEM((K, LANES), jnp.float32),
            pltpu.VMEM((ZR, LANES), jnp.float32),
            pltpu.VMEM_SHARED((NP, LANES), jnp.float32),
        ],
    )
    def deg_kernel(dst_hbm, out_hbm, idx_v, ones_v, zero_v, acc_sh):
        cid = lax.axis_index("c")
        sid = lax.axis_index("s")

        one = jnp.full((LANES,), 1.0, jnp.float32)
        zero = jnp.zeros((LANES,), jnp.float32)

        @pl.loop(0, K)
        def _(i):
            ones_v[i, :] = one

        @pl.loop(0, ZR)
        def _(i):
            zero_v[i, :] = zero

        base = sid * RPS
        for t in range(RPS // ZR):
            pltpu.sync_copy(zero_v, acc_sh.at[pl.ds(base + t * ZR, ZR)])
        pltpu.sync_copy(dst_hbm.at[sid], idx_v)
        plsc.subcore_barrier()

        lo = cid * HC

        @pl.loop(lo, lo + HC)
        def _(j):
            pltpu.sync_copy(ones_v, acc_sh.at[idx_v.at[j]], add=True)

        plsc.subcore_barrier()
        pltpu.sync_copy(acc_sh.at[pl.ds(base, RPS)],
                        out_hbm.at[cid, pl.ds(base, RPS)])

    return deg_kernel(dst3)


def _sc_aggregate(hs_split, src3, dst3, d2):
    @functools.partial(
        pl.kernel,
        out_type=jax.ShapeDtypeStruct((NC, NP, d2), jnp.float32),
        mesh=_mesh(),
        compiler_params=_SC_PARAMS,
        scratch_types=[
            pltpu.VMEM((NCHUNK, K), jnp.int32),
            pltpu.VMEM((NCHUNK, K), jnp.int32),
            pltpu.VMEM((K, d2), jnp.float32),
            pltpu.VMEM((K, d2), jnp.float32),
            pltpu.VMEM((ZR, d2), jnp.float32),
            pltpu.VMEM_SHARED((NP, d2), jnp.float32),
            pltpu.SemaphoreType.DMA,
            pltpu.SemaphoreType.DMA,
        ],
    )
    def agg_kernel(hs_hbm, src_hbm, dst_hbm, out_hbm,
                   src_v, dst_v, rows_a, rows_b, zero_v, acc_sh,
                   sem_a, sem_b):
        cid = lax.axis_index("c")
        sid = lax.axis_index("s")

        zero = jnp.zeros((LANES,), jnp.float32)

        @pl.loop(0, ZR)
        def _(i):
            @pl.loop(0, d2, step=LANES)
            def _(c0):
                zero_v[i, pl.ds(c0, LANES)] = zero

        base = sid * RPS
        for t in range(RPS // ZR):
            pltpu.sync_copy(zero_v, acc_sh.at[pl.ds(base + t * ZR, ZR)])
        pltpu.sync_copy(src_hbm.at[sid], src_v)
        pltpu.sync_copy(dst_hbm.at[sid], dst_v)
        plsc.subcore_barrier()

        hs_c = hs_hbm.at[cid]
        pltpu.async_copy(hs_c.at[src_v.at[0]], rows_a, sem_a)
        pltpu.async_copy(hs_c.at[src_v.at[1]], rows_b, sem_b)

        @pl.loop(0, NCHUNK, step=2)
        def _(j):
            pltpu.make_async_copy(hs_c.at[src_v.at[j]], rows_a, sem_a).wait()
            pltpu.sync_copy(rows_a, acc_sh.at[dst_v.at[j]], add=True)

            @pl.when(j + 2 < NCHUNK)
            def _():
                pltpu.async_copy(hs_c.at[src_v.at[j + 2]], rows_a, sem_a)

            pltpu.make_async_copy(hs_c.at[src_v.at[j + 1]],
                                  rows_b, sem_b).wait()
            pltpu.sync_copy(rows_b, acc_sh.at[dst_v.at[j + 1]], add=True)

            @pl.when(j + 3 < NCHUNK)
            def _():
                pltpu.async_copy(hs_c.at[src_v.at[j + 3]], rows_b, sem_b)

        plsc.subcore_barrier()
        pltpu.sync_copy(acc_sh.at[pl.ds(base, RPS)],
                        out_hbm.at[cid, pl.ds(base, RPS)])

    return agg_kernel(hs_split, src3, dst3)


def _dinv_of(degp_blk):
    deg = degp_blk[0] + degp_blk[1]
    return lax.rsqrt(deg[:, 0:1] + 1.0)


def _tc_matmul(x, w):
    m, k = x.shape
    n = w.shape[1]

    def body(x_ref, w_ref, o_ref):
        o_ref[...] = jnp.dot(x_ref[...], w_ref[...],
                             preferred_element_type=jnp.float32)

    return pl.pallas_call(
        body,
        grid=(m // BR,),
        in_specs=[pl.BlockSpec((BR, k), lambda i: (i, 0)),
                  pl.BlockSpec((k, n), lambda i: (0, 0))],
        out_specs=pl.BlockSpec((BR, n), lambda i: (i, 0)),
        out_shape=jax.ShapeDtypeStruct((m, n), jnp.float32),
    )(x, w)


def _tc_scale(h, degp):
    m, n = h.shape
    d2 = n // 2

    def body(h_ref, degp_ref, o_ref):
        hs = h_ref[...] * _dinv_of(degp_ref[...])
        o_ref[0] = hs[:, :d2]
        o_ref[1] = hs[:, d2:]

    return pl.pallas_call(
        body,
        grid=(m // BR,),
        in_specs=[pl.BlockSpec((BR, n), lambda i: (i, 0)),
                  pl.BlockSpec((NC, BR, LANES), lambda i: (0, i, 0))],
        out_specs=pl.BlockSpec((NC, BR, d2), lambda i: (0, i, 0)),
        out_shape=jax.ShapeDtypeStruct((NC, m, d2), jnp.float32),
    )(h, degp)


def _tc_mid(aggp, hs1s, degp, b1, w2):
    nc, m, d2 = hs1s.shape
    n = 2 * d2
    n2 = w2.shape[1]
    q2 = n2 // 2

    def body(a_ref, hs_ref, degp_ref, b_ref, w_ref, o_ref):
        dinv = _dinv_of(degp_ref[...])
        agg = jnp.concatenate([a_ref[0], a_ref[1]], axis=1)
        hs1 = jnp.concatenate([hs_ref[0], hs_ref[1]], axis=1)
        z = (agg + hs1) * dinv + b_ref[...]
        h2 = jnp.dot(jnp.maximum(z, 0.0), w_ref[...],
                     preferred_element_type=jnp.float32)
        hs2 = h2 * dinv
        o_ref[0] = hs2[:, :q2]
        o_ref[1] = hs2[:, q2:]

    return pl.pallas_call(
        body,
        grid=(m // BR,),
        in_specs=[pl.BlockSpec((NC, BR, d2), lambda i: (0, i, 0)),
                  pl.BlockSpec((NC, BR, d2), lambda i: (0, i, 0)),
                  pl.BlockSpec((NC, BR, LANES), lambda i: (0, i, 0)),
                  pl.BlockSpec((1, n), lambda i: (0, 0)),
                  pl.BlockSpec((n, n2), lambda i: (0, 0))],
        out_specs=pl.BlockSpec((NC, BR, q2), lambda i: (0, i, 0)),
        out_shape=jax.ShapeDtypeStruct((NC, m, q2), jnp.float32),
    )(aggp, hs1s, degp, b1, w2)


def _tc_fin(aggp, hs2s, degp, b2):
    nc, m, q2 = hs2s.shape
    n = 2 * q2

    def body(a_ref, hs_ref, degp_ref, b_ref, o_ref):
        dinv = _dinv_of(degp_ref[...])
        agg = jnp.concatenate([a_ref[0], a_ref[1]], axis=1)
        hs2 = jnp.concatenate([hs_ref[0], hs_ref[1]], axis=1)
        o_ref[...] = (agg + hs2) * dinv + b_ref[...]

    return pl.pallas_call(
        body,
        grid=(m // BR,),
        in_specs=[pl.BlockSpec((NC, BR, q2), lambda i: (0, i, 0)),
                  pl.BlockSpec((NC, BR, q2), lambda i: (0, i, 0)),
                  pl.BlockSpec((NC, BR, LANES), lambda i: (0, i, 0)),
                  pl.BlockSpec((1, n), lambda i: (0, 0))],
        out_specs=pl.BlockSpec((BR, n), lambda i: (i, 0)),
        out_shape=jax.ShapeDtypeStruct((m, n), jnp.float32),
    )(aggp, hs2s, degp, b2)


def kernel(x, edge_index, W1, b1, W2, b2):
    ei = edge_index.astype(jnp.int32)
    ei = jnp.pad(ei, ((0, 0), (0, EP - E)), constant_values=N)
    src = ei[0].reshape(NS, NCHUNK, K)
    dst = ei[1].reshape(NS, NCHUNK, K)
    xp = jnp.pad(x, ((0, NP - N), (0, 0)))

    degp = _sc_degree(dst)
    h1 = _tc_matmul(xp, W1)
    hs1s = _tc_scale(h1, degp)
    agg1 = _sc_aggregate(hs1s, src, dst, D_HID // 2)
    hs2s = _tc_mid(agg1, hs1s, degp, b1.reshape(1, D_HID), W2)
    agg2 = _sc_aggregate(hs2s, src, dst, D_OUT // 2)
    return _tc_fin(agg2, hs2s, degp, b2.reshape(1, D_OUT))[:N]
